# baseline (speedup 1.0000x reference)
import functools

import jax
import jax.numpy as jnp
from jax import lax
from jax.experimental import pallas as pl
from jax.experimental.pallas import tpu as pltpu
from jax.experimental.pallas import tpu_sc as plsc

N = 10000
E = 320000
NC, NS = 2, 16
NW = NC * NS
NP = 10240
EP = NW * 10240
KCH = 80
RPT = NP // NS
TCB = 1280
TCG = NP // TCB


def _leaky(v):
    return jnp.where(v >= 0, v, 0.01 * v)


def _mesh():
    return plsc.VectorSubcoreMesh(
        core_axis_name="c", subcore_axis_name="s", num_cores=NC, num_subcores=NS
    )


def _deg_body(dst_hbm, ones_hbm, zeros_hbm, out_hbm, dstv, onesv, acc, sem):
    c = lax.axis_index("c")
    s = lax.axis_index("s")
    wid = s * NC + c
    pltpu.sync_copy(zeros_hbm, acc.at[pl.ds(s * RPT, RPT), :])
    pltpu.sync_copy(ones_hbm, onesv)
    pltpu.sync_copy(dst_hbm.at[wid], dstv)
    plsc.subcore_barrier()

    def step(k, carry):
        pltpu.sync_copy(onesv, acc.at[dstv.at[k]], add=True)
        return carry

    lax.fori_loop(0, KCH, step, 0)
    plsc.subcore_barrier()
    pltpu.sync_copy(
        acc.at[pl.ds(s * RPT, RPT), :],
        out_hbm.at[c, pl.ds(s * RPT, RPT), :],
    )


_deg_kernel = functools.partial(
    pl.kernel,
    _deg_body,
    out_type=jax.ShapeDtypeStruct((NC, NP, 128), jnp.float32),
    mesh=_mesh(),
    scratch_types=[
        pltpu.VMEM((KCH, 128), jnp.int32),
        pltpu.VMEM((128, 128), jnp.float32),
        pltpu.VMEM_SHARED((NP, 128), jnp.float32),
        pltpu.SemaphoreType.DMA,
    ],
)


NBUF = 2
GRP = 16
NGRP = KCH // GRP


K0 = 48
K1 = 2 * KCH - K0


def _agg_body(W, src_hbm, dst_hbm, t_hbm, zeros_hbm, out_hbm,
              srcg, dstg, r0, r1, acc, g0, g1, ssem):
    c = lax.axis_index("c")
    s = lax.axis_index("s")
    rows = (r0, r1)
    gsem = (g0, g1)
    pltpu.sync_copy(zeros_hbm, acc.at[pl.ds(s * RPT, RPT), :])
    plsc.subcore_barrier()

    def run(base_row, nchunks):
        def group(g, carry):
            row0 = base_row + g * GRP
            pltpu.sync_copy(src_hbm.at[pl.ds(row0, GRP)], srcg)
            pltpu.sync_copy(dst_hbm.at[pl.ds(row0, GRP)], dstg)

            def step(j, carry2):
                gd = [pltpu.async_copy(t_hbm.at[srcg.at[NBUF * j + b]],
                                       rows[b], gsem[b])
                      for b in range(NBUF)]
                sd = []
                for b in range(NBUF):
                    gd[b].wait()
                    sd.append(pltpu.async_copy(rows[b],
                                               acc.at[dstg.at[NBUF * j + b]],
                                               ssem, add=True))
                for d in sd:
                    d.wait()
                return carry2

            lax.fori_loop(0, GRP // NBUF, step, 0)
            return carry

        lax.fori_loop(0, nchunks // GRP, group, 0)

    @pl.when(c == 0)
    def _():
        run(s * K0, K0)

    @pl.when(c == 1)
    def _():
        run(16 * K0 + s * K1, K1)

    plsc.subcore_barrier()
    pltpu.sync_copy(
        acc.at[pl.ds(s * RPT, RPT), :], out_hbm.at[c, pl.ds(s * RPT, RPT), :]
    )


def _agg_kernel(W):
    return functools.partial(
        pl.kernel,
        functools.partial(_agg_body, W),
        out_type=jax.ShapeDtypeStruct((NC, NP, W), jnp.float32),
        mesh=_mesh(),
        scratch_types=[
            pltpu.VMEM((GRP, 128), jnp.int32),
            pltpu.VMEM((GRP, 128), jnp.int32),
            pltpu.VMEM((128, W), jnp.float32),
            pltpu.VMEM((128, W), jnp.float32),
            pltpu.VMEM_SHARED((NP, W), jnp.float32),
            pltpu.SemaphoreType.DMA,
            pltpu.SemaphoreType.DMA,
            pltpu.SemaphoreType.DMA,
        ],
    )


def _dis(degref):
    deg = degref[0, :, :1] + degref[1, :, :1] + 1.0
    return lax.rsqrt(deg)


def _tc1_body(x_ref, w_ref, b_ref, deg_ref, o_ref):
    h = lax.dot_general(x_ref[...], w_ref[...], (((1,), (1,)), ((), ())),
                        preferred_element_type=jnp.float32)
    o_ref[...] = _dis(deg_ref) * _leaky(h + b_ref[...])


def _tc2_body(p_ref, t_ref, deg_ref, w1_ref, b1_ref, w2_ref, o_ref):
    dis = _dis(deg_ref)
    g1 = dis * (p_ref[0] + p_ref[1] + t_ref[...])
    h1 = _leaky(
        lax.dot_general(g1, w1_ref[...], (((1,), (1,)), ((), ())),
                        preferred_element_type=jnp.float32) + b1_ref[...])
    o_ref[...] = dis * lax.dot_general(
        h1, w2_ref[...], (((1,), (1,)), ((), ())),
        preferred_element_type=jnp.float32)


def _tc3_body(p_ref, t_ref, deg_ref, b2_ref, o_ref):
    dis = _dis(deg_ref)
    g2 = dis * (p_ref[0] + p_ref[1] + t_ref[...])
    o_ref[...] = dis * _leaky(g2 + b2_ref[...])


def _tc4_body(p_ref, t_ref, deg_ref, w3_ref, b3_ref, wo_ref, bo_ref, o_ref):
    dis = _dis(deg_ref)
    g3 = dis * (p_ref[0] + p_ref[1] + t_ref[...])
    u3 = lax.dot_general(g3, w3_ref[...], (((1,), (1,)), ((), ())),
                         preferred_element_type=jnp.float32) + b3_ref[...]
    h3 = _leaky(u3)
    o_ref[...] = jnp.sum(h3 * wo_ref[...], axis=1, keepdims=True) + bo_ref[...]


def _row_spec(w):
    return pl.BlockSpec((TCB, w), lambda i: (i, 0))


def _part_spec(w):
    return pl.BlockSpec((NC, TCB, w), lambda i: (0, i, 0))


def _full_spec(shape):
    nd = len(shape)
    return pl.BlockSpec(shape, lambda i: (0,) * nd)


_DEG_SPEC = pl.BlockSpec((NC, TCB, 128), lambda i: (0, i, 0))


def kernel(x, edge_index, W_in, b_in, W1, b1, W2, b2, W3, b3, W_out, b_out):
    f32 = jnp.float32
    src = edge_index[0].astype(jnp.int32)
    dst = edge_index[1].astype(jnp.int32)
    pad = jnp.full((EP - E,), N, jnp.int32)
    src3 = jnp.concatenate([src, pad]).reshape(NW, KCH, 128)
    dst3 = jnp.concatenate([dst, pad]).reshape(NW, KCH, 128)
    src2 = src3.reshape(NW * KCH, 128)
    dst2 = dst3.reshape(NW * KCH, 128)
    xp = jnp.pad(x, ((0, NP - N), (0, 0)))

    ones128 = jnp.ones((128, 128), f32)
    z128 = jnp.zeros((RPT, 128), f32)

    degP = _deg_kernel()(dst3, ones128, z128)

    th0 = pl.pallas_call(
        _tc1_body,
        grid=(TCG,),
        in_specs=[_row_spec(128), _full_spec((128, 128)),
                  _full_spec((1, 128)), _DEG_SPEC],
        out_specs=_row_spec(128),
        out_shape=jax.ShapeDtypeStruct((NP, 128), f32),
    )(xp, W_in, b_in.reshape(1, 128), degP)

    P1 = _agg_kernel(128)()(src2, dst2, th0, z128)

    tu2 = pl.pallas_call(
        _tc2_body,
        grid=(TCG,),
        in_specs=[_part_spec(128), _row_spec(128), _DEG_SPEC,
                  _full_spec((256, 128)), _full_spec((1, 256)),
                  _full_spec((128, 256))],
        out_specs=_row_spec(128),
        out_shape=jax.ShapeDtypeStruct((NP, 128), f32),
    )(P1, th0, degP, W1, b1.reshape(1, 256), W2)

    P2 = _agg_kernel(128)()(src2, dst2, tu2, z128)

    th2 = pl.pallas_call(
        _tc3_body,
        grid=(TCG,),
        in_specs=[_part_spec(128), _row_spec(128), _DEG_SPEC,
                  _full_spec((1, 128))],
        out_specs=_row_spec(128),
        out_shape=jax.ShapeDtypeStruct((NP, 128), f32),
    )(P2, tu2, degP, b2.reshape(1, 128))

    P3 = _agg_kernel(128)()(src2, dst2, th2, z128)

    y = pl.pallas_call(
        _tc4_body,
        grid=(TCG,),
        in_specs=[_part_spec(128), _row_spec(128), _DEG_SPEC,
                  _full_spec((64, 128)), _full_spec((1, 64)),
                  _full_spec((1, 64)), _full_spec((1, 1))],
        out_specs=_row_spec(1),
        out_shape=jax.ShapeDtypeStruct((NP, 1), f32),
    )(P3, th2, degP, W3, b3.reshape(1, 64), W_out, b_out.reshape(1, 1))

    return y[:N]


# speedup vs baseline: 1.1667x; 1.1667x over previous
import functools

import jax
import jax.numpy as jnp
from jax import lax
from jax.experimental import pallas as pl
from jax.experimental.pallas import tpu as pltpu
from jax.experimental.pallas import tpu_sc as plsc

N = 10000
E = 320000
NC, NS = 2, 16
NW = NC * NS
NP = 10240
EP = NW * 10240
KCH = 80
RPT = NP // NS
TCB = 1280
TCG = NP // TCB


def _leaky(v):
    return jnp.where(v >= 0, v, 0.01 * v)


def _mesh():
    return plsc.VectorSubcoreMesh(
        core_axis_name="c", subcore_axis_name="s", num_cores=NC, num_subcores=NS
    )


def _deg_body(dst_hbm, ones_hbm, zeros_hbm, out_hbm, dstv, onesv, acc, sem):
    c = lax.axis_index("c")
    s = lax.axis_index("s")
    wid = s * NC + c
    pltpu.sync_copy(zeros_hbm, acc.at[pl.ds(s * RPT, RPT), :])
    pltpu.sync_copy(ones_hbm, onesv)
    pltpu.sync_copy(dst_hbm.at[wid], dstv)
    plsc.subcore_barrier()

    def step(k, carry):
        pltpu.sync_copy(onesv, acc.at[dstv.at[k]], add=True)
        return carry

    lax.fori_loop(0, KCH, step, 0)
    plsc.subcore_barrier()
    pltpu.sync_copy(
        acc.at[pl.ds(s * RPT, RPT), :],
        out_hbm.at[c, pl.ds(s * RPT, RPT), :],
    )


_deg_kernel = functools.partial(
    pl.kernel,
    _deg_body,
    out_type=jax.ShapeDtypeStruct((NC, NP, 128), jnp.float32),
    mesh=_mesh(),
    scratch_types=[
        pltpu.VMEM((KCH, 128), jnp.int32),
        pltpu.VMEM((128, 128), jnp.float32),
        pltpu.VMEM_SHARED((NP, 128), jnp.float32),
        pltpu.SemaphoreType.DMA,
    ],
)


NBUF = 2
GRP = 16
NGRP = KCH // GRP


K0 = 112
K1 = 2 * KCH - K0


def _agg_body(W, src_hbm, dst_hbm, t_hbm, zeros_hbm, out_hbm,
              srcg, dstg, r0, r1, acc, g0, g1, ssem):
    c = lax.axis_index("c")
    s = lax.axis_index("s")
    rows = (r0, r1)
    gsem = (g0, g1)
    pltpu.sync_copy(zeros_hbm, acc.at[pl.ds(s * RPT, RPT), :])
    plsc.subcore_barrier()

    def run(base_row, nchunks):
        def group(g, carry):
            row0 = base_row + g * GRP
            pltpu.sync_copy(src_hbm.at[pl.ds(row0, GRP)], srcg)
            pltpu.sync_copy(dst_hbm.at[pl.ds(row0, GRP)], dstg)

            def step(j, carry2):
                gd = [pltpu.async_copy(t_hbm.at[srcg.at[NBUF * j + b]],
                                       rows[b], gsem[b])
                      for b in range(NBUF)]
                sd = []
                for b in range(NBUF):
                    gd[b].wait()
                    sd.append(pltpu.async_copy(rows[b],
                                               acc.at[dstg.at[NBUF * j + b]],
                                               ssem, add=True))
                for d in sd:
                    d.wait()
                return carry2

            lax.fori_loop(0, GRP // NBUF, step, 0)
            return carry

        lax.fori_loop(0, nchunks // GRP, group, 0)

    @pl.when(c == 0)
    def _():
        run(s * K0, K0)

    @pl.when(c == 1)
    def _():
        run(16 * K0 + s * K1, K1)

    plsc.subcore_barrier()
    pltpu.sync_copy(
        acc.at[pl.ds(s * RPT, RPT), :], out_hbm.at[c, pl.ds(s * RPT, RPT), :]
    )


def _agg_kernel(W):
    return functools.partial(
        pl.kernel,
        functools.partial(_agg_body, W),
        out_type=jax.ShapeDtypeStruct((NC, NP, W), jnp.float32),
        mesh=_mesh(),
        scratch_types=[
            pltpu.VMEM((GRP, 128), jnp.int32),
            pltpu.VMEM((GRP, 128), jnp.int32),
            pltpu.VMEM((128, W), jnp.float32),
            pltpu.VMEM((128, W), jnp.float32),
            pltpu.VMEM_SHARED((NP, W), jnp.float32),
            pltpu.SemaphoreType.DMA,
            pltpu.SemaphoreType.DMA,
            pltpu.SemaphoreType.DMA,
        ],
    )


def _dis(degref):
    deg = degref[0, :, :1] + degref[1, :, :1] + 1.0
    return lax.rsqrt(deg)


def _tc1_body(x_ref, w_ref, b_ref, deg_ref, o_ref):
    h = lax.dot_general(x_ref[...], w_ref[...], (((1,), (1,)), ((), ())),
                        preferred_element_type=jnp.float32)
    o_ref[...] = _dis(deg_ref) * _leaky(h + b_ref[...])


def _tc2_body(p_ref, t_ref, deg_ref, w1_ref, b1_ref, w2_ref, o_ref):
    dis = _dis(deg_ref)
    g1 = dis * (p_ref[0] + p_ref[1] + t_ref[...])
    h1 = _leaky(
        lax.dot_general(g1, w1_ref[...], (((1,), (1,)), ((), ())),
                        preferred_element_type=jnp.float32) + b1_ref[...])
    o_ref[...] = dis * lax.dot_general(
        h1, w2_ref[...], (((1,), (1,)), ((), ())),
        preferred_element_type=jnp.float32)


def _tc3_body(p_ref, t_ref, deg_ref, b2_ref, o_ref):
    dis = _dis(deg_ref)
    g2 = dis * (p_ref[0] + p_ref[1] + t_ref[...])
    o_ref[...] = dis * _leaky(g2 + b2_ref[...])


def _tc4_body(p_ref, t_ref, deg_ref, w3_ref, b3_ref, wo_ref, bo_ref, o_ref):
    dis = _dis(deg_ref)
    g3 = dis * (p_ref[0] + p_ref[1] + t_ref[...])
    u3 = lax.dot_general(g3, w3_ref[...], (((1,), (1,)), ((), ())),
                         preferred_element_type=jnp.float32) + b3_ref[...]
    h3 = _leaky(u3)
    o_ref[...] = jnp.sum(h3 * wo_ref[...], axis=1, keepdims=True) + bo_ref[...]


def _row_spec(w):
    return pl.BlockSpec((TCB, w), lambda i: (i, 0))


def _part_spec(w):
    return pl.BlockSpec((NC, TCB, w), lambda i: (0, i, 0))


def _full_spec(shape):
    nd = len(shape)
    return pl.BlockSpec(shape, lambda i: (0,) * nd)


_DEG_SPEC = pl.BlockSpec((NC, TCB, 128), lambda i: (0, i, 0))


def kernel(x, edge_index, W_in, b_in, W1, b1, W2, b2, W3, b3, W_out, b_out):
    f32 = jnp.float32
    src = edge_index[0].astype(jnp.int32)
    dst = edge_index[1].astype(jnp.int32)
    pad = jnp.full((EP - E,), N, jnp.int32)
    src3 = jnp.concatenate([src, pad]).reshape(NW, KCH, 128)
    dst3 = jnp.concatenate([dst, pad]).reshape(NW, KCH, 128)
    src2 = src3.reshape(NW * KCH, 128)
    dst2 = dst3.reshape(NW * KCH, 128)
    xp = jnp.pad(x, ((0, NP - N), (0, 0)))

    ones128 = jnp.ones((128, 128), f32)
    z128 = jnp.zeros((RPT, 128), f32)

    degP = _deg_kernel()(dst3, ones128, z128)

    th0 = pl.pallas_call(
        _tc1_body,
        grid=(TCG,),
        in_specs=[_row_spec(128), _full_spec((128, 128)),
                  _full_spec((1, 128)), _DEG_SPEC],
        out_specs=_row_spec(128),
        out_shape=jax.ShapeDtypeStruct((NP, 128), f32),
    )(xp, W_in, b_in.reshape(1, 128), degP)

    P1 = _agg_kernel(128)()(src2, dst2, th0, z128)

    tu2 = pl.pallas_call(
        _tc2_body,
        grid=(TCG,),
        in_specs=[_part_spec(128), _row_spec(128), _DEG_SPEC,
                  _full_spec((256, 128)), _full_spec((1, 256)),
                  _full_spec((128, 256))],
        out_specs=_row_spec(128),
        out_shape=jax.ShapeDtypeStruct((NP, 128), f32),
    )(P1, th0, degP, W1, b1.reshape(1, 256), W2)

    P2 = _agg_kernel(128)()(src2, dst2, tu2, z128)

    th2 = pl.pallas_call(
        _tc3_body,
        grid=(TCG,),
        in_specs=[_part_spec(128), _row_spec(128), _DEG_SPEC,
                  _full_spec((1, 128))],
        out_specs=_row_spec(128),
        out_shape=jax.ShapeDtypeStruct((NP, 128), f32),
    )(P2, tu2, degP, b2.reshape(1, 128))

    P3 = _agg_kernel(128)()(src2, dst2, th2, z128)

    y = pl.pallas_call(
        _tc4_body,
        grid=(TCG,),
        in_specs=[_part_spec(128), _row_spec(128), _DEG_SPEC,
                  _full_spec((64, 128)), _full_spec((1, 64)),
                  _full_spec((1, 64)), _full_spec((1, 1))],
        out_specs=_row_spec(1),
        out_shape=jax.ShapeDtypeStruct((NP, 1), f32),
    )(P3, th2, degP, W3, b3.reshape(1, 64), W_out, b_out.reshape(1, 1))

    return y[:N]
